# baseline (speedup 1.0000x reference)
import functools

import jax
import jax.numpy as jnp
from jax import lax
from jax.experimental import pallas as pl
from jax.experimental.pallas import tpu as pltpu
from jax.experimental.pallas import tpu_sc as plsc

_B, _N, _M, _DIM = 16, 128, 128, 64
_INF = 1000000000.0
_L = 16
_NCH = _N // _L
_PAD = 16
_ROW = _PAD + _N + _L


def _stage1_tc_kernel(x_ref, y_ref, s_out_ref):
    lane_n = jax.lax.broadcasted_iota(jnp.int32, (2 * _N, _N), 1)
    ones_row = jnp.ones((1, _DIM), jnp.float32)
    for b in range(_B):
        Xb = x_ref[b]
        Yb = y_ref[b]
        x2row = jax.lax.dot_general(
            ones_row, Xb * Xb,
            (((1,), (1,)), ((), ())),
            preferred_element_type=jnp.float32,
            precision=jax.lax.Precision.HIGHEST,
        )
        y2col = jnp.sum(Yb * Yb, axis=1, keepdims=True)
        C = jax.lax.dot_general(
            Yb, Xb,
            (((1,), (1,)), ((), ())),
            preferred_element_type=jnp.float32,
            precision=jax.lax.Precision.HIGHEST,
        )
        Db = (x2row + y2col) - 2.0 * C

        S = jnp.concatenate(
            [Db, jnp.full((2 * _N - _M, _N), _INF, jnp.float32)], axis=0)
        for k in range(7):
            bit = 1 << k
            rolled = jnp.concatenate([S[-bit:], S[:-bit]], axis=0)
            S = jnp.where((lane_n & bit) != 0, rolled, S)
        s_out_ref[b] = S


def _sc_dp_kernel(s_hbm, out_hbm, s_v, ae_v, ao_v, me_v, mo_v):
    c = lax.axis_index("c")
    s = lax.axis_index("s")
    b = c * 8 + s

    @pl.when(s < 8)
    def _():
        pltpu.sync_copy(s_hbm.at[pl.ds(b * (2 * _N * _N), 2 * _N * _N)], s_v)

        iota = lax.broadcasted_iota(jnp.int32, (_L,), 0)
        inf_vec = jnp.full((_L,), _INF, jnp.float32)

        for j in range(_ROW // _L):
            ae_v[pl.ds(j * _L, _L)] = inf_vec
            ao_v[pl.ds(j * _L, _L)] = inf_vec
            me_v[pl.ds(j * _L, _L)] = inf_vec
            mo_v[pl.ds(j * _L, _L)] = inf_vec

        start = jnp.where(iota == 0, 0.0, _INF)
        for j in range(_NCH):
            t0 = s_v[pl.ds(j * _L, _L)]
            a0 = t0 + (start if j == 0 else inf_vec)
            ae_v[pl.ds(_PAD + j * _L, _L)] = a0
            plsc.store_scatter(me_v, [iota + (_PAD + 1 + j * _L)], a0)

        def onestep(d, a_prev, m_prev, a_new, m_new):
            for j in range(_NCH):
                off = _PAD + j * _L
                t = s_v[pl.ds(d * _N + j * _L, _L)]
                a1c = a_prev[pl.ds(off, _L)]
                msh = m_prev[pl.ds(off, _L)]
                a0 = t + jnp.minimum(a1c, msh)
                a_new[pl.ds(off, _L)] = a0
                plsc.store_scatter(m_new, [iota + (off + 1)],
                                   jnp.minimum(a0, a1c))

        def body(k, carry):
            onestep(2 * k + 1, ae_v, me_v, ao_v, mo_v)
            onestep(2 * k + 2, ao_v, mo_v, ae_v, me_v)
            return carry

        lax.fori_loop(0, _N - 1, body, 0)

        pltpu.sync_copy(ae_v.at[pl.ds(_PAD, _N)], out_hbm.at[pl.ds(b * _N, _N)])


_sc_dp = functools.partial(
    pl.kernel,
    out_type=jax.ShapeDtypeStruct((_B * _N,), jnp.float32),
    mesh=plsc.VectorSubcoreMesh(core_axis_name="c", subcore_axis_name="s",
                                num_cores=2, num_subcores=16),
    scratch_types=[
        pltpu.VMEM((2 * _N * _N,), jnp.float32),
        pltpu.VMEM((_ROW,), jnp.float32),
        pltpu.VMEM((_ROW,), jnp.float32),
        pltpu.VMEM((_ROW,), jnp.float32),
        pltpu.VMEM((_ROW,), jnp.float32),
    ],
    compiler_params=pltpu.CompilerParams(needs_layout_passes=False),
)(_sc_dp_kernel)


def kernel(X, Y):
    S = pl.pallas_call(
        _stage1_tc_kernel,
        out_shape=jax.ShapeDtypeStruct((_B, 2 * _N, _N), jnp.float32),
    )(X, Y)
    out = _sc_dp(S.reshape(_B * 2 * _N * _N))
    return out.reshape(_B, _N)[:, _N - 1]


# speedup vs baseline: 1.3319x; 1.3319x over previous
import functools

import jax
import jax.numpy as jnp
from jax import lax
from jax.experimental import pallas as pl
from jax.experimental.pallas import tpu as pltpu
from jax.experimental.pallas import tpu_sc as plsc

_B, _N, _M, _DIM = 16, 128, 128, 64
_INF = 1000000000.0
_L = 16
_NCH = _N // _L
_PAD = 16
_ROW = _PAD + _N + _L


def _stage1_tc_kernel(x_ref, y_ref, s_out_ref):
    lane_n = jax.lax.broadcasted_iota(jnp.int32, (2 * _N, _N), 1)
    ones_row = jnp.ones((1, _DIM), jnp.float32)
    for b in range(_B):
        Xb = x_ref[b]
        Yb = y_ref[b]
        x2row = jax.lax.dot_general(
            ones_row, Xb * Xb,
            (((1,), (1,)), ((), ())),
            preferred_element_type=jnp.float32,
            precision=jax.lax.Precision.HIGHEST,
        )
        y2col = jnp.sum(Yb * Yb, axis=1, keepdims=True)
        C = jax.lax.dot_general(
            Yb, Xb,
            (((1,), (1,)), ((), ())),
            preferred_element_type=jnp.float32,
            precision=jax.lax.Precision.HIGHEST,
        )
        Db = (x2row + y2col) - 2.0 * C

        S = jnp.concatenate(
            [Db, jnp.full((2 * _N - _M, _N), _INF, jnp.float32)], axis=0)
        for k in range(7):
            bit = 1 << k
            rolled = jnp.concatenate([S[-bit:], S[:-bit]], axis=0)
            S = jnp.where((lane_n & bit) != 0, rolled, S)
        s_out_ref[b] = S


def _sc_dp_kernel(s_hbm, out_hbm, s_v, row_v):
    c = lax.axis_index("c")
    s = lax.axis_index("s")
    b = c * 8 + s

    @pl.when(s < 8)
    def _():
        pltpu.sync_copy(s_hbm.at[pl.ds(b * (2 * _N * _N), 2 * _N * _N)], s_v)

        iota = lax.broadcasted_iota(jnp.int32, (_L,), 0)
        inf_vec = jnp.full((_L,), _INF, jnp.float32)
        lane0 = iota == 0
        idxrot = (iota + (_L - 1)) & (_L - 1)
        dn = lax.GatherDimensionNumbers(
            offset_dims=(), collapsed_slice_dims=(0,), start_index_map=(0,))

        def rotate(v):
            return lax.gather(v, idxrot[:, None], dn, (1,),
                              mode=lax.GatherScatterMode.PROMISE_IN_BOUNDS)

        def shift_all(ms):
            rots = [rotate(m) for m in ms]
            sh = [jnp.where(lane0, inf_vec, rots[0])]
            for j in range(1, _NCH):
                sh.append(jnp.where(lane0, rots[j - 1], rots[j]))
            return sh

        start = jnp.where(lane0, 0.0, _INF)
        p1 = []
        for j in range(_NCH):
            t0 = s_v[pl.ds(j * _L, _L)]
            p1.append(t0 + (start if j == 0 else inf_vec))
        msh = shift_all(p1)

        def body(d, carry):
            p1c, mshc = carry
            a0, m = [], []
            for j in range(_NCH):
                t = s_v[pl.ds(d * _N + j * _L, _L)]
                aj = t + jnp.minimum(p1c[j], mshc[j])
                a0.append(aj)
                m.append(jnp.minimum(aj, p1c[j]))
            return tuple(a0), tuple(shift_all(m))

        p1, msh = lax.fori_loop(1, 2 * _N - 1, body, (tuple(p1), tuple(msh)))

        for j in range(_NCH):
            row_v[pl.ds(j * _L, _L)] = p1[j]
        pltpu.sync_copy(row_v, out_hbm.at[pl.ds(b * _N, _N)])


_sc_dp = functools.partial(
    pl.kernel,
    out_type=jax.ShapeDtypeStruct((_B * _N,), jnp.float32),
    mesh=plsc.VectorSubcoreMesh(core_axis_name="c", subcore_axis_name="s",
                                num_cores=2, num_subcores=16),
    scratch_types=[
        pltpu.VMEM((2 * _N * _N,), jnp.float32),
        pltpu.VMEM((_N,), jnp.float32),
    ],
    compiler_params=pltpu.CompilerParams(needs_layout_passes=False),
)(_sc_dp_kernel)


def kernel(X, Y):
    S = pl.pallas_call(
        _stage1_tc_kernel,
        out_shape=jax.ShapeDtypeStruct((_B, 2 * _N, _N), jnp.float32),
    )(X, Y)
    out = _sc_dp(S.reshape(_B * 2 * _N * _N))
    return out.reshape(_B, _N)[:, _N - 1]
